# Initial kernel scaffold; baseline (speedup 1.0000x reference)
import jax
import jax.numpy as jnp
from jax.experimental import pallas as pl


def kernel(h, edge_index, W0, al0, ar0, b0, W1, al1, ar1, b1, W2, al2, ar2, b2, resW2):
    raise NotImplementedError("write your pallas kernel here")


# speedup vs baseline: 1.0713x; 1.0713x over previous
import functools

import jax
import jax.numpy as jnp
from jax.experimental import pallas as pl

N = 10000
E = 320000
D_IN = 128
HID = 128
H0 = 8
H1 = 8
HL = 1
NCLS = 64
NEG_SLOPE = 0.2


def _matmul_kernel(x_ref, w_ref, o_ref):
    o_ref[...] = jnp.dot(x_ref[...], w_ref[...],
                         preferred_element_type=jnp.float32)


def _matmul(x, w, block_rows=1000):
    n, k = x.shape
    _, m = w.shape
    grid = n // block_rows
    return pl.pallas_call(
        _matmul_kernel,
        grid=(grid,),
        in_specs=[
            pl.BlockSpec((block_rows, k), lambda i: (i, 0)),
            pl.BlockSpec((k, m), lambda i: (0, 0)),
        ],
        out_specs=pl.BlockSpec((block_rows, m), lambda i: (i, 0)),
        out_shape=jax.ShapeDtypeStruct((n, m), jnp.float32),
    )(x, w)


def _gat_layer(x, src, dst, W, a_l, a_r, b, num_heads, out_dim,
               res=None, act=None):
    n = x.shape[0]
    feat = _matmul(x, W).reshape(n, num_heads, out_dim)
    el = jnp.sum(feat * a_l[None, :, :], axis=-1)
    er = jnp.sum(feat * a_r[None, :, :], axis=-1)
    e = jax.nn.leaky_relu(el[src] + er[dst], NEG_SLOPE)
    c = jnp.max(el) + jnp.max(er)
    p = jnp.exp(e - c)
    denom = jax.ops.segment_sum(p, dst, num_segments=n)
    alpha = p / jnp.maximum(denom[dst], 1e-12)
    msg = feat[src] * alpha[:, :, None]
    rst = jax.ops.segment_sum(msg, dst, num_segments=n)
    if res is not None:
        rst = rst + res.reshape(n, num_heads, out_dim)
    rst = rst + b.reshape(1, num_heads, out_dim)
    if act is not None:
        rst = act(rst)
    return rst


def kernel(h, edge_index, W0, al0, ar0, b0, W1, al1, ar1, b1,
           W2, al2, ar2, b2, resW2):
    src = edge_index[0]
    dst = edge_index[1]
    relu = jax.nn.relu
    x = _gat_layer(h, src, dst, W0, al0, ar0, b0, H0, HID, act=relu)
    x = x.reshape(N, -1)
    x = _gat_layer(x, src, dst, W1, al1, ar1, b1, H1, HID, res=x, act=relu)
    x = x.reshape(N, -1)
    res2 = _matmul(x, resW2)
    logits = _gat_layer(x, src, dst, W2, al2, ar2, b2, HL, NCLS, res=res2)
    logits = logits.mean(axis=1)
    logits = logits / jnp.maximum(
        jnp.linalg.norm(logits, axis=1, keepdims=True), 1e-12)
    return (x, logits)
